# baseline (speedup 1.0000x reference)
import jax
import jax.numpy as jnp
from jax import lax
from jax.experimental import pallas as pl
from jax.experimental.pallas import tpu as pltpu
from jax.experimental.pallas import tpu_sc as plsc

BATCH = 16384
D = 32
PACK = 4
W128 = 128
L = 16
NC = 2
NS = 16
NW = NC * NS
BPW = BATCH // NW
CHUNK = 128
NCHUNK = BPW // CHUNK
GPC = CHUNK // L


def _mf_body(user_hbm, item_hbm, uw_hbm, iw_hbm, out_hbm,
             uidx_v, iidx_v, uq_v, iq_v, urows_v, irows_v, out_v, sem):
    c = lax.axis_index("c")
    s = lax.axis_index("s")
    wid = s * NC + c

    pltpu.sync_copy(user_hbm.at[pl.ds(wid * NCHUNK, NCHUNK)], uidx_v)
    pltpu.sync_copy(item_hbm.at[pl.ds(wid * NCHUNK, NCHUNK)], iidx_v)

    iota = lax.iota(jnp.int32, L)

    def rnd(j, carry):
        def qgrp(i, c2):
            uvec = uidx_v[j, pl.ds(i * L, L)]
            ivec = iidx_v[j, pl.ds(i * L, L)]
            uq_v[pl.ds(i * L, L)] = lax.shift_right_logical(uvec, 2)
            iq_v[pl.ds(i * L, L)] = lax.shift_right_logical(ivec, 2)
            return c2
        lax.fori_loop(0, GPC, qgrp, 0)

        cp1 = pltpu.async_copy(uw_hbm.at[uq_v], urows_v, sem)
        cp2 = pltpu.async_copy(iw_hbm.at[iq_v], irows_v, sem)
        cp1.wait()
        cp2.wait()

        def grp(i, c2):
            rows = i * L + iota
            uvec = uidx_v[j, pl.ds(i * L, L)]
            ivec = iidx_v[j, pl.ds(i * L, L)]
            uoff = lax.shift_left(jnp.bitwise_and(uvec, 3), 5)
            ioff = lax.shift_left(jnp.bitwise_and(ivec, 3), 5)
            acc = jnp.zeros((L,), jnp.float32)
            for d in range(D):
                cu = plsc.load_gather(urows_v, [rows, uoff + d])
                cv = plsc.load_gather(irows_v, [rows, ioff + d])
                acc = acc + cu * cv
            e = jnp.exp(-jnp.abs(acc))
            p = 1.0 / (1.0 + e)
            out_v[pl.ds(j * CHUNK + i * L, L)] = jnp.where(acc >= 0, p, 1.0 - p)
            return c2
        lax.fori_loop(0, GPC, grp, 0)
        return carry

    lax.fori_loop(0, NCHUNK, rnd, 0)
    pltpu.sync_copy(out_v, out_hbm.at[pl.ds(wid * BPW, BPW)])


@jax.jit
def kernel(user, item, user_embed_w, item_embed_w):
    mesh = plsc.VectorSubcoreMesh(core_axis_name="c", subcore_axis_name="s",
                                  num_cores=NC, num_subcores=NS)
    mf = pl.kernel(
        _mf_body,
        out_type=jax.ShapeDtypeStruct((BATCH,), jnp.float32),
        mesh=mesh,
        scratch_types=[
            pltpu.VMEM((NCHUNK, CHUNK), jnp.int32),
            pltpu.VMEM((NCHUNK, CHUNK), jnp.int32),
            pltpu.VMEM((CHUNK,), jnp.int32),
            pltpu.VMEM((CHUNK,), jnp.int32),
            pltpu.VMEM((CHUNK, W128), jnp.float32),
            pltpu.VMEM((CHUNK, W128), jnp.float32),
            pltpu.VMEM((BPW,), jnp.float32),
            pltpu.SemaphoreType.DMA,
        ],
        compiler_params=pltpu.CompilerParams(
            needs_layout_passes=False, use_tc_tiling_on_sc=True),
    )
    user2d = user.reshape(NW * NCHUNK, CHUNK)
    item2d = item.reshape(NW * NCHUNK, CHUNK)
    uw_packed = user_embed_w.reshape(1000000 * D // W128, W128)
    iw_packed = item_embed_w.reshape(1000000 * D // W128, W128)
    return mf(user2d, item2d, uw_packed, iw_packed)
